# Initial kernel scaffold; baseline (speedup 1.0000x reference)
import jax
import jax.numpy as jnp
from jax.experimental import pallas as pl


def kernel(queries, keys):
    raise NotImplementedError("write your pallas kernel here")


# speedup vs baseline: 2.7934x; 2.7934x over previous
import jax
import jax.numpy as jnp
from jax import lax
from jax.experimental import pallas as pl
from jax.experimental.pallas import tpu as pltpu

_TOPK = 16
_BK = 2048
_R = _BK // 128
_EXTRA = 64
_PAD_IDX = 2**30


def _dist_block(q_ref, k_ref, q2_ref, k2_ref):
    s = lax.dot_general(
        q_ref[...].astype(jnp.bfloat16), k_ref[...].astype(jnp.bfloat16),
        (((1,), (1,)), ((), ())),
        preferred_element_type=jnp.float32,
    )
    return q2_ref[...] - 2.0 * s + k2_ref[...]


def _pass_a_body(q_ref, k_ref, q2_ref, k2_ref, cmin_ref, cidx_ref, thr_ref):
    j = pl.program_id(0)
    nq = q_ref.shape[0]

    @pl.when(j == 0)
    def _init():
        cmin_ref[...] = jnp.full(cmin_ref.shape, jnp.inf, jnp.float32)
        cidx_ref[...] = jnp.full(cidx_ref.shape, _PAD_IDX, jnp.int32)

    d3 = _dist_block(q_ref, k_ref, q2_ref, k2_ref).reshape(nq, _R, 128)
    bmin = jnp.min(d3, axis=1)
    sub = lax.broadcasted_iota(jnp.int32, d3.shape, 1)
    barg = jnp.min(jnp.where(d3 == bmin[:, None, :], sub, _R), axis=1)
    lane = lax.broadcasted_iota(jnp.int32, (nq, 128), 1)
    bidx = j * _BK + barg * 128 + lane
    upd = bmin < cmin_ref[...]
    cidx_ref[...] = jnp.where(upd, bidx, cidx_ref[...])
    cmin_ref[...] = jnp.where(upd, bmin, cmin_ref[...])

    @pl.when(j == pl.num_programs(0) - 1)
    def _thresh():
        w = cmin_ref[...]
        pos = lax.broadcasted_iota(jnp.int32, w.shape, 1)
        m = None
        for i in range(_TOPK):
            m = jnp.min(w, axis=1, keepdims=True)
            if i + 1 < _TOPK:
                sp = jnp.min(jnp.where(w == m, pos, _PAD_IDX), axis=1,
                             keepdims=True)
                w = jnp.where(pos == sp, jnp.inf, w)
        thr_ref[...] = m


def _min2(x, keepdims=True):
    return jnp.min(jnp.min(x, axis=2), axis=1, keepdims=keepdims)


def _pass_b_body(q_ref, k_ref, q2_ref, k2_ref, cmin_ref, cidx_ref, thr_ref,
                 vals_ref, idx_ref, ev_ref, ei_ref, pc_ref):
    j = pl.program_id(0)
    nq = q_ref.shape[0]

    @pl.when(j == 0)
    def _init():
        ev_ref[...] = jnp.full(ev_ref.shape, jnp.inf, jnp.float32)
        ei_ref[...] = jnp.full(ei_ref.shape, _PAD_IDX, jnp.int32)
        pc_ref[...] = jnp.zeros(pc_ref.shape, jnp.int32)

    d3 = _dist_block(q_ref, k_ref, q2_ref, k2_ref).reshape(nq, _R, 128)
    sub = lax.broadcasted_iota(jnp.int32, d3.shape, 1)
    lane3 = lax.broadcasted_iota(jnp.int32, d3.shape, 2)
    gidx3 = j * _BK + sub * 128 + lane3
    t3 = thr_ref[...][:, :, None]
    flag = (d3 <= t3) & (gidx3 != cidx_ref[...][:, None, :])
    work = jnp.where(flag, d3, jnp.inf)
    cnt = jnp.sum(jnp.sum(flag.astype(jnp.int32), axis=2), axis=1,
                  keepdims=True)
    nmax = jnp.max(cnt)
    lane64 = lax.broadcasted_iota(jnp.int32, (nq, _EXTRA), 1)

    def body(_, carry):
        work, ev, ei, p = carry
        m = _min2(work)
        valid = m < jnp.inf
        si = _min2(jnp.where(work == m[:, :, None], gidx3, _PAD_IDX))
        oh = (lane64 == p) & valid
        ev = jnp.where(oh, m, ev)
        ei = jnp.where(oh, si, ei)
        p = p + valid.astype(jnp.int32)
        work = jnp.where(gidx3 == si[:, :, None], jnp.inf, work)
        return work, ev, ei, p

    carry = (work, ev_ref[...], ei_ref[...], pc_ref[...])
    _, ev, ei, p = lax.fori_loop(0, nmax, body, carry)
    ev_ref[...] = ev
    ei_ref[...] = ei
    pc_ref[...] = p

    @pl.when(j == pl.num_programs(0) - 1)
    def _merge():
        cv = jnp.concatenate([cmin_ref[...], ev_ref[...]], axis=1)
        ci = jnp.concatenate([cidx_ref[...], ei_ref[...]], axis=1)
        for i in range(_TOPK):
            m = jnp.min(cv, axis=1, keepdims=True)
            si = jnp.min(jnp.where(cv == m, ci, _PAD_IDX), axis=1,
                         keepdims=True)
            vals_ref[:, i:i + 1] = m
            idx_ref[:, i:i + 1] = si
            if i + 1 < _TOPK:
                cv = jnp.where(ci == si, jnp.inf, cv)


def kernel(queries, keys):
    nq, d = queries.shape
    nk = keys.shape[0]
    nkb = (nk + _BK - 1) // _BK
    nkp = nkb * _BK
    kpad = jnp.pad(keys, ((0, nkp - nk), (0, 0)))
    q2 = jnp.sum(queries * queries, axis=1, keepdims=True)
    k2 = jnp.sum(kpad * kpad, axis=1)
    k2 = jnp.where(jnp.arange(nkp) < nk, k2, jnp.inf)[None, :]

    const2 = lambda shape: pl.BlockSpec(shape, lambda j: (0, 0))
    stream_specs = [
        const2((nq, d)),
        pl.BlockSpec((_BK, d), lambda j: (j, 0)),
        const2((nq, 1)),
        pl.BlockSpec((1, _BK), lambda j: (0, j)),
    ]

    cmin, cidx, thr = pl.pallas_call(
        _pass_a_body,
        grid=(nkb,),
        in_specs=stream_specs,
        out_specs=[const2((nq, 128)), const2((nq, 128)), const2((nq, 1))],
        out_shape=[
            jax.ShapeDtypeStruct((nq, 128), jnp.float32),
            jax.ShapeDtypeStruct((nq, 128), jnp.int32),
            jax.ShapeDtypeStruct((nq, 1), jnp.float32),
        ],
        compiler_params=pltpu.CompilerParams(
            dimension_semantics=("arbitrary",),
        ),
    )(queries, kpad, q2, k2)

    vals, idx = pl.pallas_call(
        _pass_b_body,
        grid=(nkb,),
        in_specs=stream_specs + [const2((nq, 128)), const2((nq, 128)),
                                 const2((nq, 1))],
        out_specs=[const2((nq, _TOPK)), const2((nq, _TOPK))],
        out_shape=[
            jax.ShapeDtypeStruct((nq, _TOPK), jnp.float32),
            jax.ShapeDtypeStruct((nq, _TOPK), jnp.int32),
        ],
        scratch_shapes=[
            pltpu.VMEM((nq, _EXTRA), jnp.float32),
            pltpu.VMEM((nq, _EXTRA), jnp.int32),
            pltpu.VMEM((nq, 1), jnp.int32),
        ],
        compiler_params=pltpu.CompilerParams(
            dimension_semantics=("arbitrary",),
        ),
    )(queries, kpad, q2, k2, cmin, cidx, thr)
    return vals, idx


# speedup vs baseline: 7.4796x; 2.6776x over previous
import jax
import jax.numpy as jnp
from jax import lax
from jax.experimental import pallas as pl
from jax.experimental.pallas import tpu as pltpu

_TOPK = 16
_BK = 2048
_R = _BK // 128
_EXTRA = 64
_PAD_IDX = 2**30


def _dist_block(qb_ref, kt_ref, q2_ref, k2_ref):
    s = lax.dot_general(
        qb_ref[...], kt_ref[...], (((1,), (0,)), ((), ())),
        preferred_element_type=jnp.float32,
    )
    return q2_ref[...] - 2.0 * s + k2_ref[...]


def _tree_min(xs):
    while len(xs) > 1:
        xs = [jnp.minimum(a, b) for a, b in zip(xs[::2], xs[1::2])] + (
            [xs[-1]] if len(xs) % 2 else [])
    return xs[0]


def _pass_a_body(qb_ref, kt_ref, q2_ref, k2_ref, cmin_ref, cidx_ref, thr_ref):
    j = pl.program_id(0)
    nq = qb_ref.shape[0]

    @pl.when(j == 0)
    def _init():
        cmin_ref[...] = jnp.full(cmin_ref.shape, jnp.inf, jnp.float32)
        cidx_ref[...] = jnp.full(cidx_ref.shape, _PAD_IDX, jnp.int32)

    d2 = _dist_block(qb_ref, kt_ref, q2_ref, k2_ref)
    sl = [d2[:, g * 128:(g + 1) * 128] for g in range(_R)]
    bmin = _tree_min(sl)
    barg = jnp.full((nq, 128), _R, jnp.int32)
    for g in reversed(range(_R)):
        barg = jnp.where(sl[g] == bmin, g, barg)
    lane = lax.broadcasted_iota(jnp.int32, (nq, 128), 1)
    bidx = j * _BK + barg * 128 + lane
    upd = bmin < cmin_ref[...]
    cidx_ref[...] = jnp.where(upd, bidx, cidx_ref[...])
    cmin_ref[...] = jnp.where(upd, bmin, cmin_ref[...])

    @pl.when(j == pl.num_programs(0) - 1)
    def _thresh():
        w = cmin_ref[...]
        m = None
        for i in range(_TOPK):
            m = jnp.min(w, axis=1, keepdims=True)
            if i + 1 < _TOPK:
                sp = jnp.min(jnp.where(w == m, lane, _PAD_IDX), axis=1,
                             keepdims=True)
                w = jnp.where(lane == sp, jnp.inf, w)
        thr_ref[...] = m


def _pass_b_body(qb_ref, kt_ref, q2_ref, k2_ref, cmin_ref, cidx_ref, thr_ref,
                 vals_ref, idx_ref, ev_ref, ei_ref, pc_ref):
    j = pl.program_id(0)
    nq = qb_ref.shape[0]

    @pl.when(j == 0)
    def _init():
        ev_ref[...] = jnp.full(ev_ref.shape, jnp.inf, jnp.float32)
        ei_ref[...] = jnp.full(ei_ref.shape, _PAD_IDX, jnp.int32)
        pc_ref[...] = jnp.zeros(pc_ref.shape, jnp.int32)

    d2 = _dist_block(qb_ref, kt_ref, q2_ref, k2_ref)
    t = thr_ref[...]
    cidx = cidx_ref[...]
    lane = lax.broadcasted_iota(jnp.int32, (nq, 128), 1)

    wv = []
    cl = jnp.zeros((nq, 128), jnp.int32)
    for g in range(_R):
        dg = d2[:, g * 128:(g + 1) * 128]
        fl = (dg <= t) & ((j * _BK + g * 128 + lane) != cidx)
        wv.append(jnp.where(fl, dg, jnp.inf))
        cl = cl + fl.astype(jnp.int32)
    fmin = _tree_min(wv)
    fsub = jnp.full((nq, 128), _R, jnp.int32)
    for g in reversed(range(_R)):
        fsub = jnp.where(wv[g] == fmin, g, fsub)
    fidx = j * _BK + fsub * 128 + lane
    cl01 = jnp.minimum(cl, 1)
    cn = jnp.sum(cl01, axis=1, keepdims=True)
    res = jnp.sum(cl - cl01, axis=1, keepdims=True)
    nmax = jnp.max(cn)
    nres = jnp.max(res)
    lane64 = lax.broadcasted_iota(jnp.int32, (nq, _EXTRA), 1)

    def fast(_, carry):
        fmin, ev, ei, p = carry
        m = jnp.min(fmin, axis=1, keepdims=True)
        valid = m < jnp.inf
        si = jnp.min(jnp.where(fmin == m, fidx, _PAD_IDX), axis=1,
                     keepdims=True)
        oh = (lane64 == p) & valid
        ev = jnp.where(oh, m, ev)
        ei = jnp.where(oh, si, ei)
        p = p + valid.astype(jnp.int32)
        fmin = jnp.where(fidx == si, jnp.inf, fmin)
        return fmin, ev, ei, p

    carry = lax.fori_loop(
        0, nmax, fast, (fmin, ev_ref[...], ei_ref[...], pc_ref[...]))
    _, ev, ei, p = carry

    def slow(_, carry):
        wres, ev, ei, p = carry
        fm2 = _tree_min(list(wres))
        m = jnp.min(fm2, axis=1, keepdims=True)
        valid = m < jnp.inf
        gg = jnp.full((nq, 128), _R, jnp.int32)
        for g in reversed(range(_R)):
            gg = jnp.where(wres[g] == fm2, g, gg)
        idx2 = j * _BK + gg * 128 + lane
        si = jnp.min(jnp.where(fm2 == m, idx2, _PAD_IDX), axis=1,
                     keepdims=True)
        oh = (lane64 == p) & valid
        ev = jnp.where(oh, m, ev)
        ei = jnp.where(oh, si, ei)
        p = p + valid.astype(jnp.int32)
        wres = tuple(
            jnp.where((j * _BK + g * 128 + lane) == si, jnp.inf, wres[g])
            for g in range(_R))
        return wres, ev, ei, p

    wres0 = tuple(jnp.where(fsub == g, jnp.inf, wv[g]) for g in range(_R))
    _, ev, ei, p = lax.fori_loop(0, nres, slow, (wres0, ev, ei, p))

    ev_ref[...] = ev
    ei_ref[...] = ei
    pc_ref[...] = p

    @pl.when(j == pl.num_programs(0) - 1)
    def _merge():
        cv = jnp.concatenate([cmin_ref[...], ev_ref[...]], axis=1)
        ci = jnp.concatenate([cidx_ref[...], ei_ref[...]], axis=1)
        for i in range(_TOPK):
            m = jnp.min(cv, axis=1, keepdims=True)
            si = jnp.min(jnp.where(cv == m, ci, _PAD_IDX), axis=1,
                         keepdims=True)
            vals_ref[:, i:i + 1] = m
            idx_ref[:, i:i + 1] = si
            if i + 1 < _TOPK:
                cv = jnp.where(ci == si, jnp.inf, cv)


def kernel(queries, keys):
    nq, d = queries.shape
    nk = keys.shape[0]
    nkb = (nk + _BK - 1) // _BK
    nkp = nkb * _BK
    kpad = jnp.pad(keys, ((0, nkp - nk), (0, 0)))
    q2 = jnp.sum(queries * queries, axis=1, keepdims=True)
    k2 = jnp.sum(kpad * kpad, axis=1)
    k2 = jnp.where(jnp.arange(nkp) < nk, k2, jnp.inf)[None, :]
    qb = queries.astype(jnp.bfloat16)
    kt = kpad.astype(jnp.bfloat16).T

    const2 = lambda shape: pl.BlockSpec(shape, lambda j: (0, 0))
    stream_specs = [
        const2((nq, d)),
        pl.BlockSpec((d, _BK), lambda j: (0, j)),
        const2((nq, 1)),
        pl.BlockSpec((1, _BK), lambda j: (0, j)),
    ]

    cmin, cidx, thr = pl.pallas_call(
        _pass_a_body,
        grid=(nkb,),
        in_specs=stream_specs,
        out_specs=[const2((nq, 128)), const2((nq, 128)), const2((nq, 1))],
        out_shape=[
            jax.ShapeDtypeStruct((nq, 128), jnp.float32),
            jax.ShapeDtypeStruct((nq, 128), jnp.int32),
            jax.ShapeDtypeStruct((nq, 1), jnp.float32),
        ],
        compiler_params=pltpu.CompilerParams(
            dimension_semantics=("arbitrary",),
        ),
    )(qb, kt, q2, k2)

    vals, idx = pl.pallas_call(
        _pass_b_body,
        grid=(nkb,),
        in_specs=stream_specs + [const2((nq, 128)), const2((nq, 128)),
                                 const2((nq, 1))],
        out_specs=[const2((nq, _TOPK)), const2((nq, _TOPK))],
        out_shape=[
            jax.ShapeDtypeStruct((nq, _TOPK), jnp.float32),
            jax.ShapeDtypeStruct((nq, _TOPK), jnp.int32),
        ],
        scratch_shapes=[
            pltpu.VMEM((nq, _EXTRA), jnp.float32),
            pltpu.VMEM((nq, _EXTRA), jnp.int32),
            pltpu.VMEM((nq, 1), jnp.int32),
        ],
        compiler_params=pltpu.CompilerParams(
            dimension_semantics=("arbitrary",),
        ),
    )(qb, kt, q2, k2, cmin, cidx, thr)
    return vals, idx


# speedup vs baseline: 8.9324x; 1.1942x over previous
import jax
import jax.numpy as jnp
from jax import lax
from jax.experimental import pallas as pl
from jax.experimental.pallas import tpu as pltpu

_TOPK = 16
_BK = 2048
_R = _BK // 128
_EXTRA = 64
_PAD_IDX = 2**30


def _dist_block(qb_ref, kt_ref, q2_ref, k2_ref):
    s = lax.dot_general(
        qb_ref[...], kt_ref[...], (((1,), (0,)), ((), ())),
        preferred_element_type=jnp.float32,
    )
    return q2_ref[...] - 2.0 * s + k2_ref[...]


def _tree_min(xs):
    while len(xs) > 1:
        xs = [jnp.minimum(a, b) for a, b in zip(xs[::2], xs[1::2])] + (
            [xs[-1]] if len(xs) % 2 else [])
    return xs[0]


def _pass_a_body(qb_ref, kt_ref, q2_ref, k2_ref, cmin_ref, cidx_ref, thr_ref):
    j = pl.program_id(0)
    nq = qb_ref.shape[0]

    @pl.when(j == 0)
    def _init():
        cmin_ref[...] = jnp.full(cmin_ref.shape, jnp.inf, jnp.float32)
        cidx_ref[...] = jnp.full(cidx_ref.shape, _PAD_IDX, jnp.int32)

    d2 = _dist_block(qb_ref, kt_ref, q2_ref, k2_ref)
    sl = [d2[:, g * 128:(g + 1) * 128] for g in range(_R)]
    bmin = _tree_min(sl)
    barg = jnp.full((nq, 128), _R, jnp.int32)
    for g in reversed(range(_R)):
        barg = jnp.where(sl[g] == bmin, g, barg)
    lane = lax.broadcasted_iota(jnp.int32, (nq, 128), 1)
    bidx = j * _BK + barg * 128 + lane
    upd = bmin < cmin_ref[...]
    cidx_ref[...] = jnp.where(upd, bidx, cidx_ref[...])
    cmin_ref[...] = jnp.where(upd, bmin, cmin_ref[...])

    @pl.when(j == pl.num_programs(0) - 1)
    def _thresh():
        w = cmin_ref[...]
        m = None
        for i in range(_TOPK):
            m = jnp.min(w, axis=1, keepdims=True)
            if i + 1 < _TOPK:
                sp = jnp.min(jnp.where(w == m, lane, _PAD_IDX), axis=1,
                             keepdims=True)
                w = jnp.where(lane == sp, jnp.inf, w)
        thr_ref[...] = m


def _pass_b_body(qb_ref, kt_ref, q2_ref, k2_ref, cmin_ref, cidx_ref, thr_ref,
                 vals_ref, idx_ref, ev_ref, ei_ref, pc_ref, wres_ref):
    j = pl.program_id(0)
    nq = qb_ref.shape[0]

    @pl.when(j == 0)
    def _init():
        ev_ref[...] = jnp.full(ev_ref.shape, jnp.inf, jnp.float32)
        ei_ref[...] = jnp.full(ei_ref.shape, _PAD_IDX, jnp.int32)
        pc_ref[...] = jnp.zeros(pc_ref.shape, jnp.int32)

    d2 = _dist_block(qb_ref, kt_ref, q2_ref, k2_ref)
    t = thr_ref[...]
    cidx = cidx_ref[...]
    lane = lax.broadcasted_iota(jnp.int32, (nq, 128), 1)

    wv = []
    cl = jnp.zeros((nq, 128), jnp.int32)
    for g in range(_R):
        dg = d2[:, g * 128:(g + 1) * 128]
        fl = (dg <= t) & ((j * _BK + g * 128 + lane) != cidx)
        wv.append(jnp.where(fl, dg, jnp.inf))
        cl = cl + fl.astype(jnp.int32)
    fmin = _tree_min(wv)
    fsub = jnp.full((nq, 128), _R, jnp.int32)
    for g in reversed(range(_R)):
        fsub = jnp.where(wv[g] == fmin, g, fsub)
    fidx = j * _BK + fsub * 128 + lane
    cn = jnp.sum((fmin < jnp.inf).astype(jnp.int32), axis=1, keepdims=True)
    res = jnp.sum(cl, axis=1, keepdims=True) - cn
    nmax = jnp.max(cn)
    nres = jnp.max(res)
    lane64 = lax.broadcasted_iota(jnp.int32, (nq, _EXTRA), 1)

    def fast(_, fmin):
        m = jnp.min(fmin, axis=1, keepdims=True)
        valid = m < jnp.inf
        si = jnp.min(jnp.where(fmin == m, fidx, _PAD_IDX), axis=1,
                     keepdims=True)
        p = pc_ref[...]
        oh = (lane64 == p) & valid
        ev_ref[...] = jnp.where(oh, m, ev_ref[...])
        ei_ref[...] = jnp.where(oh, si, ei_ref[...])
        pc_ref[...] = p + valid.astype(jnp.int32)
        return jnp.where(fidx == si, jnp.inf, fmin)

    lax.fori_loop(0, nmax, fast, fmin)

    @pl.when(nres > 0)
    def _residuals():
        for g in range(_R):
            wres_ref[:, g * 128:(g + 1) * 128] = jnp.where(
                fsub == g, jnp.inf, wv[g])

        def slow(_, __):
            wr = [wres_ref[:, g * 128:(g + 1) * 128] for g in range(_R)]
            fm2 = _tree_min(list(wr))
            m = jnp.min(fm2, axis=1, keepdims=True)
            valid = m < jnp.inf
            gg = jnp.full((nq, 128), _R, jnp.int32)
            for g in reversed(range(_R)):
                gg = jnp.where(wr[g] == fm2, g, gg)
            idx2 = j * _BK + gg * 128 + lane
            si = jnp.min(jnp.where(fm2 == m, idx2, _PAD_IDX), axis=1,
                         keepdims=True)
            p = pc_ref[...]
            oh = (lane64 == p) & valid
            ev_ref[...] = jnp.where(oh, m, ev_ref[...])
            ei_ref[...] = jnp.where(oh, si, ei_ref[...])
            pc_ref[...] = p + valid.astype(jnp.int32)
            for g in range(_R):
                wres_ref[:, g * 128:(g + 1) * 128] = jnp.where(
                    (j * _BK + g * 128 + lane) == si, jnp.inf, wr[g])
            return 0

        lax.fori_loop(0, nres, slow, 0)

    @pl.when(j == pl.num_programs(0) - 1)
    def _merge():
        cv = jnp.concatenate([cmin_ref[...], ev_ref[...]], axis=1)
        ci = jnp.concatenate([cidx_ref[...], ei_ref[...]], axis=1)
        for i in range(_TOPK):
            m = jnp.min(cv, axis=1, keepdims=True)
            si = jnp.min(jnp.where(cv == m, ci, _PAD_IDX), axis=1,
                         keepdims=True)
            vals_ref[:, i:i + 1] = m
            idx_ref[:, i:i + 1] = si
            if i + 1 < _TOPK:
                cv = jnp.where(ci == si, jnp.inf, cv)


def kernel(queries, keys):
    nq, d = queries.shape
    nk = keys.shape[0]
    nkb = (nk + _BK - 1) // _BK
    nkp = nkb * _BK
    q2 = jnp.sum(queries * queries, axis=1, keepdims=True)
    k2 = jnp.concatenate(
        [jnp.sum(keys * keys, axis=1),
         jnp.full((nkp - nk,), jnp.inf, jnp.float32)])[None, :]
    qb = queries.astype(jnp.bfloat16)
    kt = jnp.pad(keys.astype(jnp.bfloat16), ((0, nkp - nk), (0, 0))).T

    const2 = lambda shape: pl.BlockSpec(shape, lambda j: (0, 0))
    stream_specs = [
        const2((nq, d)),
        pl.BlockSpec((d, _BK), lambda j: (0, j)),
        const2((nq, 1)),
        pl.BlockSpec((1, _BK), lambda j: (0, j)),
    ]

    cmin, cidx, thr = pl.pallas_call(
        _pass_a_body,
        grid=(nkb,),
        in_specs=stream_specs,
        out_specs=[const2((nq, 128)), const2((nq, 128)), const2((nq, 1))],
        out_shape=[
            jax.ShapeDtypeStruct((nq, 128), jnp.float32),
            jax.ShapeDtypeStruct((nq, 128), jnp.int32),
            jax.ShapeDtypeStruct((nq, 1), jnp.float32),
        ],
        compiler_params=pltpu.CompilerParams(
            dimension_semantics=("arbitrary",),
        ),
    )(qb, kt, q2, k2)

    vals, idx = pl.pallas_call(
        _pass_b_body,
        grid=(nkb,),
        in_specs=stream_specs + [const2((nq, 128)), const2((nq, 128)),
                                 const2((nq, 1))],
        out_specs=[const2((nq, _TOPK)), const2((nq, _TOPK))],
        out_shape=[
            jax.ShapeDtypeStruct((nq, _TOPK), jnp.float32),
            jax.ShapeDtypeStruct((nq, _TOPK), jnp.int32),
        ],
        scratch_shapes=[
            pltpu.VMEM((nq, _EXTRA), jnp.float32),
            pltpu.VMEM((nq, _EXTRA), jnp.int32),
            pltpu.VMEM((nq, 1), jnp.int32),
            pltpu.VMEM((nq, _BK), jnp.float32),
        ],
        compiler_params=pltpu.CompilerParams(
            dimension_semantics=("arbitrary",),
        ),
    )(qb, kt, q2, k2, cmin, cidx, thr)
    return vals, idx
